# Initial kernel scaffold; baseline (speedup 1.0000x reference)
import jax
import jax.numpy as jnp
from jax.experimental import pallas as pl


def kernel(x, edge_index, W1, b1, W2, b2, Wg, att_src, att_dst, bg, Wo, bo):
    raise NotImplementedError("write your pallas kernel here")


# speedup vs baseline: 21.7657x; 21.7657x over previous
import functools

import jax
import jax.numpy as jnp
from jax import lax
from jax.experimental import pallas as pl
from jax.experimental.pallas import tpu as pltpu, tpu_sc as plsc

N = 10000
E = 320000
D = 128
H0, H1, H2 = 12, 9, 7
HEADS = 8

NT = 32
NPT = 320
NPAD = NT * NPT
CH = 2560
NCH = E // CH
NBCH = CH // 16
CAP = 16384
LV = CAP + 160
GRP = 128
NEG = -3.0e38

_SC_PARAMS = pltpu.CompilerParams(needs_layout_passes=False,
                                  use_tc_tiling_on_sc=False)


def _prefix16(v):
    io = lax.iota(jnp.int32, 16)
    for k in (1, 2, 4, 8):
        sh = jnp.take(v, jnp.maximum(io - k, 0))
        v = v + jnp.where(io >= k, sh, 0)
    return v


def _splat_i(x):
    return jnp.full((16,), x, jnp.int32)


def _row(ref2d, r):
    io = lax.iota(jnp.int32, 16)
    return plsc.load_gather(ref2d, [_splat_i(r), io])


def _scan_edges(key_c, val_c, list_v, base, cnt0):
    dl = key_c - base
    m = (dl >= 0) & (dl < NPT)
    pk = (dl << 14) | val_c
    cs = _prefix16(jnp.where(m, 1, 0))
    cntc = jnp.minimum(cnt0, CAP - 16)
    pos = jnp.where(m, cntc + cs - 1, LV - 1)
    plsc.store_scatter(list_v, [pos], pk)
    return cnt0 + cs[15]


def _pad_list(list_v, cnt):
    io = lax.iota(jnp.int32, 16)
    dummy = jnp.full((16,), NPT << 14, jnp.int32)
    cntc = jnp.minimum(cnt, CAP - 16)
    for k in range(8):
        plsc.store_scatter(list_v, [cntc + k * 16 + io], dummy)
    return (cntc + 127) & ~127


def _stream_scan(key_hbm, val_hbm, kchunk, vchunk, list_v, base):
    def chunk_body(ch, cnt):
        pltpu.sync_copy(key_hbm.at[pl.ds(ch * CH, CH)], kchunk)
        pltpu.sync_copy(val_hbm.at[pl.ds(ch * CH, CH)], vchunk)

        def batch_body(i, c):
            kc = kchunk[pl.ds(i * 16, 16)]
            vc = vchunk[pl.ds(i * 16, 16)]
            return _scan_edges(kc, vc, list_v, base, c)

        return lax.fori_loop(0, NBCH, batch_body, cnt)

    cnt = lax.fori_loop(0, NCH, chunk_body, 0)
    return _pad_list(list_v, cnt), cnt


def _gather_group(q_hbm, list_v, g, gidx, qrows, sem):
    def unpack(i, _):
        pkb = list_v[pl.ds(g * GRP + i * 16, 16)]
        gidx[pl.ds(i * 16, 16)] = pkb & 16383
        return 0
    lax.fori_loop(0, GRP // 16, unpack, 0)
    pltpu.async_copy(q_hbm.at[gidx], qrows, sem).wait()


def _layer_edges(list_v, ngroups, q_hbm, gidx, qrows, sem, pstage, acc_s,
                 acc_mx, acc_mn):
    io = lax.iota(jnp.int32, 16)

    def group_body(g, _):
        _gather_group(q_hbm, list_v, g, gidx, qrows, sem)

        def batch_body(i, _2):
            pkb = list_v[pl.ds(g * GRP + i * 16, 16)]
            for j in range(16):
                dsc = pkb[j] >> 14
                rb = dsc * 16
                qv = plsc.load_gather(qrows, [_splat_i(i * 16 + j), io])
                pv = pstage[pl.ds(rb, 16)]
                mrow = pv + qv
                acc_s[pl.ds(rb, 16)] = acc_s[pl.ds(rb, 16)] + mrow
                acc_mx[pl.ds(rb, 16)] = jnp.maximum(acc_mx[pl.ds(rb, 16)], mrow)
                acc_mn[pl.ds(rb, 16)] = jnp.minimum(acc_mn[pl.ds(rb, 16)], mrow)
            return 0

        lax.fori_loop(0, GRP // 16, batch_body, 0)
        return 0

    lax.fori_loop(0, ngroups, group_body, 0)


def _write_raw48(acc_s, acc_mx, acc_mn, h2d):
    io = lax.iota(jnp.int32, 16)

    def node_body(r, _):
        rb = r * 16
        plsc.store_scatter(h2d, [_splat_i(r), io], acc_s[pl.ds(rb, 16)])
        plsc.store_scatter(h2d, [_splat_i(r), 16 + io], acc_mx[pl.ds(rb, 16)])
        plsc.store_scatter(h2d, [_splat_i(r), 32 + io], acc_mn[pl.ds(rb, 16)])
        return 0

    lax.fori_loop(0, NPT, node_body, 0)


def _init_accs(pstage, p_hbm, base, acc_s, acc_mx, acc_mn, p2d):
    pltpu.sync_copy(p_hbm.at[pl.ds(base, NPT)], p2d)
    io = lax.iota(jnp.int32, 16)

    def cp(r, _):
        pstage[pl.ds(r * 16, 16)] = plsc.load_gather(p2d, [_splat_i(r), io])
        return 0
    lax.fori_loop(0, NPT, cp, 0)

    z = jnp.zeros((16,), jnp.float32)
    hi = jnp.full((16,), -NEG, jnp.float32)
    lo = jnp.full((16,), NEG, jnp.float32)

    def zr(r, _):
        acc_s[pl.ds(r * 16, 16)] = z
        acc_mx[pl.ds(r * 16, 16)] = lo
        acc_mn[pl.ds(r * 16, 16)] = hi
        return 0
    lax.fori_loop(0, NPT + 1, zr, 0)


def _write_count(counts_hbm, cbuf, wid, cnt):
    cbuf[pl.ds(0, 16)] = _splat_i(cnt)
    pltpu.sync_copy(cbuf, counts_hbm.at[wid])


def _read_count(counts_hbm, cbuf, wid):
    pltpu.sync_copy(counts_hbm.at[wid], cbuf)
    return cbuf[pl.ds(0, 16)][0]


def _wid():
    return lax.axis_index("s") * 2 + lax.axis_index("c")


def _sc_layer1():
    mesh = plsc.VectorSubcoreMesh(core_axis_name="c", subcore_axis_name="s")

    @functools.partial(
        pl.kernel,
        out_type=(
            jax.ShapeDtypeStruct((NPAD, 48), jnp.float32),
            jax.ShapeDtypeStruct((NT, LV), jnp.int32),
            jax.ShapeDtypeStruct((NT, 16), jnp.int32),
        ),
        mesh=mesh,
        compiler_params=_SC_PARAMS,
        scratch_types=[
            pltpu.VMEM((LV,), jnp.int32),
            pltpu.VMEM((CH,), jnp.int32),
            pltpu.VMEM((CH,), jnp.int32),
            pltpu.VMEM((NPT * 16,), jnp.float32),
            pltpu.VMEM((NPT, 16), jnp.float32),
            pltpu.VMEM(((NPT + 1) * 16,), jnp.float32),
            pltpu.VMEM(((NPT + 1) * 16,), jnp.float32),
            pltpu.VMEM(((NPT + 1) * 16,), jnp.float32),
            pltpu.VMEM((GRP,), jnp.int32),
            pltpu.VMEM((GRP, 16), jnp.float32),
            pltpu.VMEM((NPT, 48), jnp.float32),
            pltpu.VMEM((16,), jnp.int32),
            pltpu.SemaphoreType.DMA,
        ],
    )
    def k(dst_hbm, src_hbm, p_hbm, q_hbm, h_hbm, lists_hbm, counts_hbm,
          list_v, kchunk, vchunk, pstage, p2d, acc_s, acc_mx, acc_mn,
          gidx, qrows, h2d, cbuf, sem):
        wid = _wid()
        base = wid * NPT
        cnt128, cnt = _stream_scan(dst_hbm, src_hbm, kchunk, vchunk, list_v,
                                   base)
        _write_count(counts_hbm, cbuf, wid, cnt)
        pltpu.sync_copy(list_v, lists_hbm.at[wid])
        _init_accs(pstage, p_hbm, base, acc_s, acc_mx, acc_mn, p2d)
        _layer_edges(list_v, cnt128 >> 7, q_hbm, gidx, qrows, sem, pstage,
                     acc_s, acc_mx, acc_mn)
        _write_raw48(acc_s, acc_mx, acc_mn, h2d)
        pltpu.sync_copy(h2d, h_hbm.at[pl.ds(base, NPT)])

    return k


def _sc_layer2():
    mesh = plsc.VectorSubcoreMesh(core_axis_name="c", subcore_axis_name="s")

    @functools.partial(
        pl.kernel,
        out_type=jax.ShapeDtypeStruct((NPAD, 48), jnp.float32),
        mesh=mesh,
        compiler_params=_SC_PARAMS,
        scratch_types=[
            pltpu.VMEM((LV,), jnp.int32),
            pltpu.VMEM((NPT * 16,), jnp.float32),
            pltpu.VMEM((NPT, 16), jnp.float32),
            pltpu.VMEM(((NPT + 1) * 16,), jnp.float32),
            pltpu.VMEM(((NPT + 1) * 16,), jnp.float32),
            pltpu.VMEM(((NPT + 1) * 16,), jnp.float32),
            pltpu.VMEM((GRP,), jnp.int32),
            pltpu.VMEM((GRP, 16), jnp.float32),
            pltpu.VMEM((NPT, 48), jnp.float32),
            pltpu.VMEM((16,), jnp.int32),
            pltpu.SemaphoreType.DMA,
        ],
    )
    def k(lists_hbm, counts_hbm, p_hbm, q_hbm, h_hbm,
          list_v, pstage, p2d, acc_s, acc_mx, acc_mn, gidx, qrows,
          h2d, cbuf, sem):
        wid = _wid()
        base = wid * NPT
        pltpu.sync_copy(lists_hbm.at[wid], list_v)
        cnt = _read_count(counts_hbm, cbuf, wid)
        cnt128 = _pad_list(list_v, cnt)
        _init_accs(pstage, p_hbm, base, acc_s, acc_mx, acc_mn, p2d)
        _layer_edges(list_v, cnt128 >> 7, q_hbm, gidx, qrows, sem, pstage,
                     acc_s, acc_mx, acc_mn)
        _write_raw48(acc_s, acc_mx, acc_mn, h2d)
        pltpu.sync_copy(h2d, h_hbm.at[pl.ds(base, NPT)])

    return k


def _head_map(c):
    g = lax.iota(jnp.int32, 16) + c * 16
    return jnp.minimum((g * 2341) >> 14, 7)


def _sc_gat():
    mesh = plsc.VectorSubcoreMesh(core_axis_name="c", subcore_axis_name="s")

    @functools.partial(
        pl.kernel,
        out_type=(
            jax.ShapeDtypeStruct((NPAD, 64), jnp.float32),
            jax.ShapeDtypeStruct((NPAD, 16), jnp.float32),
        ),
        mesh=mesh,
        compiler_params=_SC_PARAMS,
        scratch_types=[
            pltpu.VMEM((LV,), jnp.int32),
            pltpu.VMEM((CH,), jnp.int32),
            pltpu.VMEM((CH,), jnp.int32),
            pltpu.VMEM((NPT, 16), jnp.float32),
            pltpu.VMEM((NPT, 16), jnp.float32),
            pltpu.VMEM((NPT, 64), jnp.float32),
            pltpu.VMEM(((NPT + 1) * 16,), jnp.float32),
            pltpu.VMEM(((NPT + 1) * 16,), jnp.float32),
            pltpu.VMEM(((NPT + 1) * 64,), jnp.float32),
            pltpu.VMEM((GRP,), jnp.int32),
            pltpu.VMEM((GRP, 16), jnp.float32),
            pltpu.VMEM((GRP, 64), jnp.float32),
            pltpu.VMEM((NPT, 64), jnp.float32),
            pltpu.VMEM((NPT, 16), jnp.float32),
            pltpu.SemaphoreType.DMA,
        ],
    )
    def k(src_hbm, dst_hbm, as2_hbm, ad2_hbm, es2_hbm, hg_hbm, g_hbm,
          den_hbm, list_v, kchunk, vchunk, ad2d, es2d, hgown, emax, den,
          gacc, gidx, asrows, hgrows, gstage, dstage, sem):
        wid = _wid()
        base = wid * NPT
        io = lax.iota(jnp.int32, 16)

        cnt128, _cnt = _stream_scan(src_hbm, dst_hbm, kchunk, vchunk, list_v,
                                    base)
        ngroups = cnt128 >> 7

        pltpu.sync_copy(ad2_hbm.at[pl.ds(base, NPT)], ad2d)
        pltpu.sync_copy(es2_hbm.at[pl.ds(base, NPT)], es2d)
        pltpu.sync_copy(hg_hbm.at[pl.ds(base, NPT)], hgown)

        def init_emax(r, _):
            emax[pl.ds(r * 16, 16)] = _row(es2d, r)
            return 0
        lax.fori_loop(0, NPT, init_emax, 0)
        emax[pl.ds(NPT * 16, 16)] = jnp.full((16,), NEG, jnp.float32)

        def p1_group(g, _):
            _gather_group(as2_hbm, list_v, g, gidx, asrows, sem)

            def p1_batch(i, _2):
                pkb = list_v[pl.ds(g * GRP + i * 16, 16)]
                for j in range(16):
                    il = pkb[j] >> 14
                    rb = il * 16
                    av = plsc.load_gather(asrows, [_splat_i(i * 16 + j), io])
                    ev = av + _row(ad2d, il)
                    el = jnp.where(ev > 0, ev, 0.2 * ev)
                    emax[pl.ds(rb, 16)] = jnp.maximum(emax[pl.ds(rb, 16)], el)
                return 0

            lax.fori_loop(0, GRP // 16, p1_batch, 0)
            return 0

        lax.fori_loop(0, ngroups, p1_group, 0)

        hmaps = [_head_map(c) for c in range(4)]

        def init_self(r, _):
            al = jnp.exp(_row(es2d, r) - emax[pl.ds(r * 16, 16)])
            den[pl.ds(r * 16, 16)] = al
            for c in range(4):
                ah = jnp.take(al, hmaps[c])
                hv = plsc.load_gather(hgown, [_splat_i(r), c * 16 + io])
                gacc[pl.ds(r * 64 + c * 16, 16)] = ah * hv
            return 0
        lax.fori_loop(0, NPT, init_self, 0)
        den[pl.ds(NPT * 16, 16)] = jnp.full((16,), 1.0, jnp.float32)

        def p2_group(g, _):
            _gather_group(as2_hbm, list_v, g, gidx, asrows, sem)
            pltpu.async_copy(hg_hbm.at[gidx], hgrows, sem).wait()

            def p2_batch(i, _2):
                pkb = list_v[pl.ds(g * GRP + i * 16, 16)]
                for j in range(16):
                    il = pkb[j] >> 14
                    rb = il * 16
                    av = plsc.load_gather(asrows, [_splat_i(i * 16 + j), io])
                    ev = av + _row(ad2d, il)
                    el = jnp.where(ev > 0, ev, 0.2 * ev)
                    al = jnp.exp(el - emax[pl.ds(rb, 16)])
                    den[pl.ds(rb, 16)] = den[pl.ds(rb, 16)] + al
                    gb = il * 64
                    for c in range(4):
                        ah = jnp.take(al, hmaps[c])
                        hv = plsc.load_gather(
                            hgrows, [_splat_i(i * 16 + j), c * 16 + io])
                        o = pl.ds(gb + c * 16, 16)
                        gacc[o] = gacc[o] + ah * hv
                return 0

            lax.fori_loop(0, GRP // 16, p2_batch, 0)
            return 0

        lax.fori_loop(0, ngroups, p2_group, 0)

        def fin(r, _):
            plsc.store_scatter(dstage, [_splat_i(r), io],
                               den[pl.ds(r * 16, 16)])
            for c in range(4):
                gv = gacc[pl.ds(r * 64 + c * 16, 16)]
                plsc.store_scatter(gstage, [_splat_i(r), c * 16 + io], gv)
            return 0
        lax.fori_loop(0, NPT, fin, 0)
        pltpu.sync_copy(gstage, g_hbm.at[pl.ds(base, NPT)])
        pltpu.sync_copy(dstage, den_hbm.at[pl.ds(base, NPT)])

    return k


_RB = 1024


def _tc_elu(v):
    return jnp.where(v > 0, v, jnp.exp(jnp.minimum(v, 0.0)) - 1.0)


def _vrsp_post(hb):
    s = hb[:, 0:16]
    mxv = hb[:, 16:32]
    mnv = hb[:, 32:48]
    cnt = s[:, 12:13]
    has = cnt > 0
    mean = jnp.where(has, s / jnp.maximum(cnt, 1.0), 0.0)
    mxv = jnp.where(has, mxv, 0.0)
    mnv = jnp.where(has, mnv, 0.0)
    return jnp.concatenate([_tc_elu(mxv), _tc_elu(mean), _tc_elu(mnv)],
                           axis=1)


def _tc_matmul2(w, bias, pre=False):
    kdim = w.shape[0]

    def body(x_ref, w_ref, b_ref, p_ref, q_ref):
        xb = x_ref[...]
        if pre:
            xb = _vrsp_post(xb)
        y = jnp.dot(xb, w_ref[...],
                    preferred_element_type=jnp.float32,
                    precision=lax.Precision.HIGHEST) + b_ref[0:1, :]
        p_ref[...] = y[:, :16]
        q_ref[...] = y[:, 16:32]

    f = pl.pallas_call(
        body,
        grid=(NPAD // _RB,),
        in_specs=[
            pl.BlockSpec((_RB, kdim), lambda i: (i, 0)),
            pl.BlockSpec((kdim, 32), lambda i: (0, 0)),
            pl.BlockSpec((8, 32), lambda i: (0, 0)),
        ],
        out_specs=[
            pl.BlockSpec((_RB, 16), lambda i: (i, 0)),
            pl.BlockSpec((_RB, 16), lambda i: (i, 0)),
        ],
        out_shape=[
            jax.ShapeDtypeStruct((NPAD, 16), jnp.float32),
            jax.ShapeDtypeStruct((NPAD, 16), jnp.float32),
        ],
    )
    return lambda x: f(x, w, bias)


def _tc_gat_prep(wgcat, msrc, mdst):
    def body(h_ref, wg_ref, ms_ref, md_ref, hg_ref, as_ref, ad_ref, es_ref):
        hg = jnp.dot(_vrsp_post(h_ref[...]), wg_ref[...],
                     preferred_element_type=jnp.float32,
                     precision=lax.Precision.HIGHEST)
        a_s = jnp.dot(hg, ms_ref[...], preferred_element_type=jnp.float32,
                      precision=lax.Precision.HIGHEST)
        a_d = jnp.dot(hg, md_ref[...], preferred_element_type=jnp.float32,
                      precision=lax.Precision.HIGHEST)
        as2 = jnp.concatenate([a_s, a_s], axis=1)
        ad2 = jnp.concatenate([a_d, a_d], axis=1)
        ev = as2 + ad2
        hg_ref[...] = hg
        as_ref[...] = as2
        ad_ref[...] = ad2
        es_ref[...] = jnp.where(ev > 0, ev, 0.2 * ev)

    f = pl.pallas_call(
        body,
        grid=(NPAD // _RB,),
        in_specs=[
            pl.BlockSpec((_RB, 48), lambda i: (i, 0)),
            pl.BlockSpec((48, 64), lambda i: (0, 0)),
            pl.BlockSpec((64, 8), lambda i: (0, 0)),
            pl.BlockSpec((64, 8), lambda i: (0, 0)),
        ],
        out_specs=[
            pl.BlockSpec((_RB, 64), lambda i: (i, 0)),
            pl.BlockSpec((_RB, 16), lambda i: (i, 0)),
            pl.BlockSpec((_RB, 16), lambda i: (i, 0)),
            pl.BlockSpec((_RB, 16), lambda i: (i, 0)),
        ],
        out_shape=[
            jax.ShapeDtypeStruct((NPAD, 64), jnp.float32),
            jax.ShapeDtypeStruct((NPAD, 16), jnp.float32),
            jax.ShapeDtypeStruct((NPAD, 16), jnp.float32),
            jax.ShapeDtypeStruct((NPAD, 16), jnp.float32),
        ],
    )
    return lambda h2: f(h2, wgcat, msrc, mdst)


def _tc_final(p8to64, bg64, wo64, bo):
    def body(g_ref, d_ref, p_ref, bg_ref, wo_ref, bo_ref, o_ref):
        dh = jnp.dot(d_ref[...][:, 0:8], p_ref[...],
                     preferred_element_type=jnp.float32,
                     precision=lax.Precision.HIGHEST)
        y = g_ref[...] / dh + bg_ref[0:1, :]
        y = jnp.where(y > 0, y, jnp.exp(jnp.minimum(y, 0.0)) - 1.0)
        o_ref[...] = jnp.dot(y, wo_ref[...],
                             preferred_element_type=jnp.float32,
                             precision=lax.Precision.HIGHEST) + bo_ref[0:1, :]

    f = pl.pallas_call(
        body,
        grid=(NPAD // _RB,),
        in_specs=[
            pl.BlockSpec((_RB, 64), lambda i: (i, 0)),
            pl.BlockSpec((_RB, 16), lambda i: (i, 0)),
            pl.BlockSpec((8, 64), lambda i: (0, 0)),
            pl.BlockSpec((8, 64), lambda i: (0, 0)),
            pl.BlockSpec((64, 8), lambda i: (0, 0)),
            pl.BlockSpec((8, 8), lambda i: (0, 0)),
        ],
        out_specs=pl.BlockSpec((_RB, 8), lambda i: (i, 0)),
        out_shape=jax.ShapeDtypeStruct((NPAD, 8), jnp.float32),
    )
    return lambda g, den: f(g, den, p8to64, bg64, wo64, bo)


def kernel(x, edge_index, W1, b1, W2, b2, Wg, att_src, att_dst, bg, Wo, bo):
    f32 = jnp.float32
    src = edge_index[0]
    dst = edge_index[1]

    x_pad = jnp.zeros((NPAD, D), f32).at[:N].set(x)

    w1cat = jnp.zeros((D, 32), f32)
    w1cat = w1cat.at[:, 0:H0].set(W1[:D])
    w1cat = w1cat.at[:, 16:16 + H0].set(W1[D:])
    bias1 = jnp.zeros((8, 32), f32).at[0, 0:H0].set(b1).at[0, 12].set(1.0)

    w2cat = jnp.zeros((48, 32), f32)
    for blk in range(3):
        rows = W2[blk * H0:(blk + 1) * H0]
        w2cat = w2cat.at[blk * 16:blk * 16 + H0, 0:H1].set(rows[:, :])
    w2s = W2[3 * H0:]
    for blk in range(3):
        rows = w2s[blk * H0:(blk + 1) * H0]
        w2cat = w2cat.at[blk * 16:blk * 16 + H0, 16:16 + H1].set(rows[:, :])
    bias2 = jnp.zeros((8, 32), f32).at[0, 0:H1].set(b2).at[0, 12].set(1.0)

    wgcat = jnp.zeros((48, 64), f32)
    for blk in range(3):
        rows = Wg[blk * H1:(blk + 1) * H1]
        wgcat = wgcat.at[blk * 16:blk * 16 + H1, 0:HEADS * H2].set(rows)
    msrc = jnp.zeros((64, 8), f32)
    mdst = jnp.zeros((64, 8), f32)
    for h in range(HEADS):
        msrc = msrc.at[h * H2:(h + 1) * H2, h].set(att_src[h])
        mdst = mdst.at[h * H2:(h + 1) * H2, h].set(att_dst[h])

    bg64 = jnp.zeros((8, 64), f32).at[0, :HEADS * H2].set(bg)
    wo64 = jnp.zeros((64, 8), f32).at[:HEADS * H2, 0].set(Wo[:, 0])
    bo8 = jnp.zeros((8, 8), f32).at[0, 0].set(bo[0])
    hmap = jnp.minimum(jnp.arange(64) // H2, HEADS - 1)
    p8to64 = (jnp.arange(8)[:, None] == hmap[None, :]).astype(f32)

    p1, q1 = _tc_matmul2(w1cat, bias1)(x_pad)
    h1, lists, counts = _sc_layer1()(dst, src, p1, q1)

    p2, q2 = _tc_matmul2(w2cat, bias2, pre=True)(h1)
    h2 = _sc_layer2()(lists, counts, p2, q2)

    hg, as2, ad2, es2 = _tc_gat_prep(wgcat, msrc, mdst)(h2)
    g, den = _sc_gat()(src, dst, as2, ad2, es2, hg)

    out = _tc_final(p8to64, bg64, wo64, bo8)(g, den)
    return out[:N, :1]


# speedup vs baseline: 50.4001x; 2.3156x over previous
import functools

import jax
import jax.numpy as jnp
from jax import lax
from jax.experimental import pallas as pl
from jax.experimental.pallas import tpu as pltpu, tpu_sc as plsc

N = 10000
E = 320000
D = 128
H0, H1, H2 = 12, 9, 7
HEADS = 8

NT = 32
NPT = 320
NPAD = NT * NPT
CH = 2560
NCH = E // CH
NBCH = CH // 16
CAP = 16384
LV = CAP + 160
GRP = 128
NEG = -3.0e38

SUBCAP = 640
EPT = E // NT
CHD = 2000
NCHD = EPT // CHD
NBD = CHD // 16
RAWLEN = NT * NT * SUBCAP

_SC_PARAMS = pltpu.CompilerParams(needs_layout_passes=False,
                                  use_tc_tiling_on_sc=False)


def _prefix16(v):
    io = lax.iota(jnp.int32, 16)
    for k in (1, 2, 4, 8):
        sh = jnp.take(v, jnp.maximum(io - k, 0))
        v = v + jnp.where(io >= k, sh, 0)
    return v


def _splat_i(x):
    return jnp.full((16,), x, jnp.int32)


def _row(ref2d, r):
    io = lax.iota(jnp.int32, 16)
    return plsc.load_gather(ref2d, [_splat_i(r), io])


def _scan_edges(key_c, val_c, list_v, base, cnt0):
    dl = key_c - base
    m = (dl >= 0) & (dl < NPT)
    pk = (dl << 14) | val_c
    cs = _prefix16(jnp.where(m, 1, 0))
    cntc = jnp.minimum(cnt0, CAP - 16)
    pos = jnp.where(m, cntc + cs - 1, LV - 1)
    plsc.store_scatter(list_v, [pos], pk)
    return cnt0 + cs[15]


def _pad_list(list_v, cnt):
    io = lax.iota(jnp.int32, 16)
    dummy = jnp.full((16,), NPT << 14, jnp.int32)
    cntc = jnp.minimum(cnt, CAP - 16)
    for k in range(8):
        plsc.store_scatter(list_v, [cntc + k * 16 + io], dummy)
    return (cntc + 127) & ~127


def _prefmax16(v):
    io = lax.iota(jnp.int32, 16)
    for k in (1, 2, 4, 8):
        sh = jnp.take(v, jnp.maximum(io - k, 0))
        v = jnp.maximum(v, jnp.where(io >= k, sh, 0))
    return v


def _sc_distribute(interpret=False):
    mesh = plsc.VectorSubcoreMesh(core_axis_name="c", subcore_axis_name="s")

    @functools.partial(
        pl.kernel,
        out_type=(
            jax.ShapeDtypeStruct((RAWLEN,), jnp.int32),
            jax.ShapeDtypeStruct((NT * 48,), jnp.int32),
        ),
        mesh=mesh,
        interpret=interpret,
        compiler_params=_SC_PARAMS,
        scratch_types=[
            pltpu.VMEM((CHD,), jnp.int32),
            pltpu.VMEM((CHD,), jnp.int32),
            pltpu.VMEM((NT * SUBCAP,), jnp.int32),
            pltpu.VMEM((48,), jnp.int32),
            pltpu.VMEM((48,), jnp.int32),
        ],
    )
    def k(key_hbm, val_hbm, raw_hbm, cnts_hbm, kchunk, vchunk, sub, ctr,
          pcnt):
        wid = _wid()
        estart = wid * EPT
        io = lax.iota(jnp.int32, 16)
        for c in range(3):
            ctr[pl.ds(c * 16, 16)] = jnp.zeros((16,), jnp.int32)

        def chunk_body(ch, _):
            pltpu.sync_copy(key_hbm.at[pl.ds(estart + ch * CHD, CHD)], kchunk)
            pltpu.sync_copy(val_hbm.at[pl.ds(estart + ch * CHD, CHD)], vchunk)

            def batch_body(i, _2):
                d = kchunk[pl.ds(i * 16, 16)]
                s = vchunk[pl.ds(i * 16, 16)]
                b = ((d >> 6) * 205) >> 10
                pk = ((d - b * NPT) << 14) | s
                bs16, pks = plsc.sort_key_val((b << 4) | io, pk)
                bs = bs16 >> 4
                prev = jnp.take(bs, jnp.maximum(io - 1, 0))
                isfirst = (io == 0) | (bs != prev)
                sf = _prefmax16(jnp.where(isfirst, io, 0))
                rank = io - sf
                ctrv = plsc.load_gather(ctr, [bs])
                pos = jnp.minimum(ctrv + rank, SUBCAP - 1)
                nxt = jnp.take(bs, jnp.minimum(io + 1, 15))
                islast = (io == 15) | (bs != nxt)
                plsc.store_scatter(ctr, [jnp.where(islast, bs, 40)], pos + 1)
                plsc.store_scatter(sub, [bs * SUBCAP + pos], pks)
                return 0

            lax.fori_loop(0, NBD, batch_body, 0)
            return 0

        lax.fori_loop(0, NCHD, chunk_body, 0)

        dummy = jnp.full((16,), NPT << 14, jnp.int32)
        c0 = ctr[pl.ds(0, 16)]
        c1 = ctr[pl.ds(16, 16)]
        for bb in range(NT):
            cbs = jnp.minimum((c0 if bb < 16 else c1)[bb % 16], SUBCAP - 16)
            plsc.store_scatter(sub, [bb * SUBCAP + cbs + io], dummy)
            pad = (cbs + 15) & ~15
            plsc.store_scatter(pcnt, [_splat_i(bb)], _splat_i(pad))

        pltpu.sync_copy(sub, raw_hbm.at[pl.ds(wid * NT * SUBCAP,
                                              NT * SUBCAP)])
        pltpu.sync_copy(pcnt, cnts_hbm.at[pl.ds(wid * 48, 48)])

    return k


def _merge_lists(raw_hbm, cnts_hbm, wid, c2d, staging, list_v, sem):
    io = lax.iota(jnp.int32, 16)
    pltpu.sync_copy(cnts_hbm, c2d)
    c16a = plsc.load_gather(c2d, [io * 48 + _splat_i(wid)])
    c16b = plsc.load_gather(c2d, [(io + 16) * 48 + _splat_i(wid)])
    pa = _prefix16(c16a)
    pb = _prefix16(c16b)
    tot_a = pa[15]
    sa = pa - c16a
    sb = pb - c16b + tot_a
    total = tot_a + pb[15]

    for t in range(NT):
        pltpu.sync_copy(raw_hbm.at[t, pl.ds(wid * SUBCAP, SUBCAP)],
                        staging.at[t])

    for t in range(NT):
        cnt_t = (c16a if t < 16 else c16b)[t % 16]
        st = (sa if t < 16 else sb)[t % 16]

        def cp(i, _, t=t, st=st):
            v = plsc.load_gather(staging, [_splat_i(t), i * 16 + io])
            list_v[pl.ds(st + i * 16, 16)] = v
            return 0

        lax.fori_loop(0, cnt_t >> 4, cp, 0)
    return total


def _stream_scan(key_hbm, val_hbm, kchunk, vchunk, list_v, base):
    def chunk_body(ch, cnt):
        pltpu.sync_copy(key_hbm.at[pl.ds(ch * CH, CH)], kchunk)
        pltpu.sync_copy(val_hbm.at[pl.ds(ch * CH, CH)], vchunk)

        def batch_body(i, c):
            kc = kchunk[pl.ds(i * 16, 16)]
            vc = vchunk[pl.ds(i * 16, 16)]
            return _scan_edges(kc, vc, list_v, base, c)

        return lax.fori_loop(0, NBCH, batch_body, cnt)

    cnt = lax.fori_loop(0, NCH, chunk_body, 0)
    return _pad_list(list_v, cnt), cnt


def _unpack_grp(list_v, g, gidx, halfoff):
    def unpack(i, _):
        pkb = list_v[pl.ds(g * GRP + i * 16, 16)]
        gidx[pl.ds(halfoff + i * 16, 16)] = pkb & 16383
        return 0
    lax.fori_loop(0, GRP // 16, unpack, 0)


def _fire_gathers(g, gidx, pairs, sem0, sem1):
    @pl.when((g & 1) == 0)
    def _():
        for (src, dstv) in pairs:
            pltpu.async_copy(src.at[gidx.at[pl.ds(0, GRP)]],
                             dstv.at[pl.ds(0, GRP)], sem0)

    @pl.when((g & 1) == 1)
    def _():
        for (src, dstv) in pairs:
            pltpu.async_copy(src.at[gidx.at[pl.ds(GRP, GRP)]],
                             dstv.at[pl.ds(GRP, GRP)], sem1)


def _drain_gathers(g, pairs, sem0, sem1):
    @pl.when((g & 1) == 0)
    def _():
        for (src, dstv) in pairs:
            pltpu.make_async_copy(src.at[pl.ds(0, GRP)],
                                  dstv.at[pl.ds(0, GRP)], sem0).wait()

    @pl.when((g & 1) == 1)
    def _():
        for (src, dstv) in pairs:
            pltpu.make_async_copy(src.at[pl.ds(0, GRP)],
                                  dstv.at[pl.ds(GRP, GRP)], sem1).wait()


def _db_prologue(list_v, ngroups, gidx, pairs, sem0):
    @pl.when(ngroups > 0)
    def _():
        _unpack_grp(list_v, 0, gidx, 0)
        for (src, dstv) in pairs:
            pltpu.async_copy(src.at[gidx.at[pl.ds(0, GRP)]],
                             dstv.at[pl.ds(0, GRP)], sem0)


def _layer_edges(list_v, ngroups, q_hbm, gidx, qrows, sem0, sem1, pstage,
                 acc_s, acc_mx, acc_mn):
    io = lax.iota(jnp.int32, 16)
    pairs = [(q_hbm, qrows)]
    _db_prologue(list_v, ngroups, gidx, pairs, sem0)

    def group_body(g, _):
        @pl.when(g + 1 < ngroups)
        def _():
            _unpack_grp(list_v, g + 1, gidx, ((g + 1) & 1) * GRP)
            _fire_gathers(g + 1, gidx, pairs, sem0, sem1)
        _drain_gathers(g, pairs, sem0, sem1)
        qoff = (g & 1) * GRP

        def batch_body(i, _2):
            pkb = list_v[pl.ds(g * GRP + i * 16, 16)]
            for j in range(16):
                dsc = pkb[j] >> 14
                rb = dsc * 16
                qv = plsc.load_gather(qrows,
                                      [_splat_i(qoff + i * 16 + j), io])
                pv = pstage[pl.ds(rb, 16)]
                mrow = pv + qv
                acc_s[pl.ds(rb, 16)] = acc_s[pl.ds(rb, 16)] + mrow
                acc_mx[pl.ds(rb, 16)] = jnp.maximum(acc_mx[pl.ds(rb, 16)], mrow)
                acc_mn[pl.ds(rb, 16)] = jnp.minimum(acc_mn[pl.ds(rb, 16)], mrow)
            return 0

        lax.fori_loop(0, GRP // 16, batch_body, 0)
        return 0

    lax.fori_loop(0, ngroups, group_body, 0)


def _write_raw48(acc_s, acc_mx, acc_mn, h2d):
    io = lax.iota(jnp.int32, 16)

    def node_body(r, _):
        rb = r * 16
        plsc.store_scatter(h2d, [_splat_i(r), io], acc_s[pl.ds(rb, 16)])
        plsc.store_scatter(h2d, [_splat_i(r), 16 + io], acc_mx[pl.ds(rb, 16)])
        plsc.store_scatter(h2d, [_splat_i(r), 32 + io], acc_mn[pl.ds(rb, 16)])
        return 0

    lax.fori_loop(0, NPT, node_body, 0)


def _init_accs(pstage, p_hbm, base, acc_s, acc_mx, acc_mn, p2d):
    pltpu.sync_copy(p_hbm.at[pl.ds(base, NPT)], p2d)
    io = lax.iota(jnp.int32, 16)

    def cp(r, _):
        pstage[pl.ds(r * 16, 16)] = plsc.load_gather(p2d, [_splat_i(r), io])
        return 0
    lax.fori_loop(0, NPT, cp, 0)

    z = jnp.zeros((16,), jnp.float32)
    hi = jnp.full((16,), -NEG, jnp.float32)
    lo = jnp.full((16,), NEG, jnp.float32)

    def zr(r, _):
        acc_s[pl.ds(r * 16, 16)] = z
        acc_mx[pl.ds(r * 16, 16)] = lo
        acc_mn[pl.ds(r * 16, 16)] = hi
        return 0
    lax.fori_loop(0, NPT + 1, zr, 0)


def _write_count(counts_hbm, cbuf, wid, cnt):
    cbuf[pl.ds(0, 16)] = _splat_i(cnt)
    pltpu.sync_copy(cbuf, counts_hbm.at[wid])


def _read_count(counts_hbm, cbuf, wid):
    pltpu.sync_copy(counts_hbm.at[wid], cbuf)
    return cbuf[pl.ds(0, 16)][0]


def _wid():
    return lax.axis_index("s") * 2 + lax.axis_index("c")


def _sc_layer1():
    mesh = plsc.VectorSubcoreMesh(core_axis_name="c", subcore_axis_name="s")

    @functools.partial(
        pl.kernel,
        out_type=(
            jax.ShapeDtypeStruct((NPAD, 48), jnp.float32),
            jax.ShapeDtypeStruct((NT, LV), jnp.int32),
            jax.ShapeDtypeStruct((NT, 16), jnp.int32),
        ),
        mesh=mesh,
        compiler_params=_SC_PARAMS,
        scratch_types=[
            pltpu.VMEM((LV,), jnp.int32),
            pltpu.VMEM((NT * 48,), jnp.int32),
            pltpu.VMEM((NT, SUBCAP), jnp.int32),
            pltpu.VMEM((NPT * 16,), jnp.float32),
            pltpu.VMEM((NPT, 16), jnp.float32),
            pltpu.VMEM(((NPT + 1) * 16,), jnp.float32),
            pltpu.VMEM(((NPT + 1) * 16,), jnp.float32),
            pltpu.VMEM(((NPT + 1) * 16,), jnp.float32),
            pltpu.VMEM((2 * GRP,), jnp.int32),
            pltpu.VMEM((2 * GRP, 16), jnp.float32),
            pltpu.VMEM((NPT, 48), jnp.float32),
            pltpu.VMEM((16,), jnp.int32),
            pltpu.SemaphoreType.DMA,
            pltpu.SemaphoreType.DMA,
        ],
    )
    def k(raw_hbm, cnts_hbm, p_hbm, q_hbm, h_hbm, lists_hbm, counts_hbm,
          list_v, c2d, staging, pstage, p2d, acc_s, acc_mx, acc_mn,
          gidx, qrows, h2d, cbuf, sem0, sem1):
        wid = _wid()
        base = wid * NPT
        cnt = _merge_lists(raw_hbm, cnts_hbm, wid, c2d, staging, list_v, sem0)
        cnt128 = _pad_list(list_v, cnt)
        _write_count(counts_hbm, cbuf, wid, cnt)
        pltpu.sync_copy(list_v, lists_hbm.at[wid])
        _init_accs(pstage, p_hbm, base, acc_s, acc_mx, acc_mn, p2d)
        _layer_edges(list_v, cnt128 >> 7, q_hbm, gidx, qrows, sem0, sem1,
                     pstage, acc_s, acc_mx, acc_mn)
        _write_raw48(acc_s, acc_mx, acc_mn, h2d)
        pltpu.sync_copy(h2d, h_hbm.at[pl.ds(base, NPT)])

    return k


def _sc_layer2():
    mesh = plsc.VectorSubcoreMesh(core_axis_name="c", subcore_axis_name="s")

    @functools.partial(
        pl.kernel,
        out_type=jax.ShapeDtypeStruct((NPAD, 48), jnp.float32),
        mesh=mesh,
        compiler_params=_SC_PARAMS,
        scratch_types=[
            pltpu.VMEM((LV,), jnp.int32),
            pltpu.VMEM((NPT * 16,), jnp.float32),
            pltpu.VMEM((NPT, 16), jnp.float32),
            pltpu.VMEM(((NPT + 1) * 16,), jnp.float32),
            pltpu.VMEM(((NPT + 1) * 16,), jnp.float32),
            pltpu.VMEM(((NPT + 1) * 16,), jnp.float32),
            pltpu.VMEM((2 * GRP,), jnp.int32),
            pltpu.VMEM((2 * GRP, 16), jnp.float32),
            pltpu.VMEM((NPT, 48), jnp.float32),
            pltpu.VMEM((16,), jnp.int32),
            pltpu.SemaphoreType.DMA,
            pltpu.SemaphoreType.DMA,
        ],
    )
    def k(lists_hbm, counts_hbm, p_hbm, q_hbm, h_hbm,
          list_v, pstage, p2d, acc_s, acc_mx, acc_mn, gidx, qrows,
          h2d, cbuf, sem0, sem1):
        wid = _wid()
        base = wid * NPT
        pltpu.sync_copy(lists_hbm.at[wid], list_v)
        cnt = _read_count(counts_hbm, cbuf, wid)
        cnt128 = _pad_list(list_v, cnt)
        _init_accs(pstage, p_hbm, base, acc_s, acc_mx, acc_mn, p2d)
        _layer_edges(list_v, cnt128 >> 7, q_hbm, gidx, qrows, sem0, sem1,
                     pstage, acc_s, acc_mx, acc_mn)
        _write_raw48(acc_s, acc_mx, acc_mn, h2d)
        pltpu.sync_copy(h2d, h_hbm.at[pl.ds(base, NPT)])

    return k


def _head_map(c):
    g = lax.iota(jnp.int32, 16) + c * 16
    return jnp.minimum((g * 2341) >> 14, 7)


def _sc_gat():
    mesh = plsc.VectorSubcoreMesh(core_axis_name="c", subcore_axis_name="s")

    @functools.partial(
        pl.kernel,
        out_type=(
            jax.ShapeDtypeStruct((NPAD * 64,), jnp.float32),
            jax.ShapeDtypeStruct((NPAD * 16,), jnp.float32),
        ),
        mesh=mesh,
        compiler_params=_SC_PARAMS,
        scratch_types=[
            pltpu.VMEM((LV,), jnp.int32),
            pltpu.VMEM((NT * 48,), jnp.int32),
            pltpu.VMEM((NT, SUBCAP), jnp.int32),
            pltpu.VMEM((NPT, 16), jnp.float32),
            pltpu.VMEM((NPT, 16), jnp.float32),
            pltpu.VMEM((NPT, 64), jnp.float32),
            pltpu.VMEM(((NPT + 1) * 16,), jnp.float32),
            pltpu.VMEM(((NPT + 1) * 16,), jnp.float32),
            pltpu.VMEM(((NPT + 1) * 64,), jnp.float32),
            pltpu.VMEM((2 * GRP,), jnp.int32),
            pltpu.VMEM((2 * GRP, 16), jnp.float32),
            pltpu.VMEM((2 * GRP, 64), jnp.float32),
            pltpu.SemaphoreType.DMA,
            pltpu.SemaphoreType.DMA,
        ],
    )
    def k(raw_hbm, cnts_hbm, as2_hbm, ad2_hbm, es2_hbm, hg_hbm, g_hbm,
          den_hbm, list_v, c2d, staging, ad2d, es2d, hgown, emax, den,
          gacc, gidx, asrows, hgrows, sem0, sem1):
        wid = _wid()
        base = wid * NPT
        io = lax.iota(jnp.int32, 16)

        cnt = _merge_lists(raw_hbm, cnts_hbm, wid, c2d, staging, list_v, sem0)
        cnt128 = _pad_list(list_v, cnt)
        ngroups = cnt128 >> 7

        pltpu.sync_copy(ad2_hbm.at[pl.ds(base, NPT)], ad2d)
        pltpu.sync_copy(es2_hbm.at[pl.ds(base, NPT)], es2d)
        pltpu.sync_copy(hg_hbm.at[pl.ds(base, NPT)], hgown)

        def init_emax(r, _):
            emax[pl.ds(r * 16, 16)] = _row(es2d, r)
            return 0
        lax.fori_loop(0, NPT, init_emax, 0)
        emax[pl.ds(NPT * 16, 16)] = jnp.full((16,), NEG, jnp.float32)

        pairs1 = [(as2_hbm, asrows)]
        _db_prologue(list_v, ngroups, gidx, pairs1, sem0)

        def p1_group(g, _):
            @pl.when(g + 1 < ngroups)
            def _():
                _unpack_grp(list_v, g + 1, gidx, ((g + 1) & 1) * GRP)
                _fire_gathers(g + 1, gidx, pairs1, sem0, sem1)
            _drain_gathers(g, pairs1, sem0, sem1)
            aoff = (g & 1) * GRP

            def p1_batch(i, _2):
                pkb = list_v[pl.ds(g * GRP + i * 16, 16)]
                for j in range(16):
                    il = pkb[j] >> 14
                    rb = il * 16
                    av = plsc.load_gather(asrows,
                                          [_splat_i(aoff + i * 16 + j), io])
                    ev = av + _row(ad2d, il)
                    el = jnp.where(ev > 0, ev, 0.2 * ev)
                    emax[pl.ds(rb, 16)] = jnp.maximum(emax[pl.ds(rb, 16)], el)
                return 0

            lax.fori_loop(0, GRP // 16, p1_batch, 0)
            return 0

        lax.fori_loop(0, ngroups, p1_group, 0)

        hmaps = [_head_map(c) for c in range(4)]

        def init_self(r, _):
            al = jnp.exp(_row(es2d, r) - emax[pl.ds(r * 16, 16)])
            den[pl.ds(r * 16, 16)] = al
            for c in range(4):
                ah = jnp.take(al, hmaps[c])
                hv = plsc.load_gather(hgown, [_splat_i(r), c * 16 + io])
                gacc[pl.ds(r * 64 + c * 16, 16)] = ah * hv
            return 0
        lax.fori_loop(0, NPT, init_self, 0)
        den[pl.ds(NPT * 16, 16)] = jnp.full((16,), 1.0, jnp.float32)

        pairs2 = [(as2_hbm, asrows), (hg_hbm, hgrows)]
        _db_prologue(list_v, ngroups, gidx, pairs2, sem0)

        def p2_group(g, _):
            @pl.when(g + 1 < ngroups)
            def _():
                _unpack_grp(list_v, g + 1, gidx, ((g + 1) & 1) * GRP)
                _fire_gathers(g + 1, gidx, pairs2, sem0, sem1)
            _drain_gathers(g, pairs2, sem0, sem1)
            aoff = (g & 1) * GRP

            def p2_batch(i, _2):
                pkb = list_v[pl.ds(g * GRP + i * 16, 16)]
                for j in range(16):
                    il = pkb[j] >> 14
                    rb = il * 16
                    av = plsc.load_gather(asrows,
                                          [_splat_i(aoff + i * 16 + j), io])
                    ev = av + _row(ad2d, il)
                    el = jnp.where(ev > 0, ev, 0.2 * ev)
                    al = jnp.exp(el - emax[pl.ds(rb, 16)])
                    den[pl.ds(rb, 16)] = den[pl.ds(rb, 16)] + al
                    gb = il * 64
                    for c in range(4):
                        ah = jnp.take(al, hmaps[c])
                        hv = plsc.load_gather(
                            hgrows, [_splat_i(aoff + i * 16 + j), c * 16 + io])
                        o = pl.ds(gb + c * 16, 16)
                        gacc[o] = gacc[o] + ah * hv
                return 0

            lax.fori_loop(0, GRP // 16, p2_batch, 0)
            return 0

        lax.fori_loop(0, ngroups, p2_group, 0)

        pltpu.sync_copy(gacc.at[pl.ds(0, NPT * 64)],
                        g_hbm.at[pl.ds(base * 64, NPT * 64)])
        pltpu.sync_copy(den.at[pl.ds(0, NPT * 16)],
                        den_hbm.at[pl.ds(base * 16, NPT * 16)])

    return k


_RB = 1024


def _tc_elu(v):
    return jnp.where(v > 0, v, jnp.exp(jnp.minimum(v, 0.0)) - 1.0)


def _vrsp_post(hb):
    s = hb[:, 0:16]
    mxv = hb[:, 16:32]
    mnv = hb[:, 32:48]
    cnt = s[:, 12:13]
    has = cnt > 0
    mean = jnp.where(has, s / jnp.maximum(cnt, 1.0), 0.0)
    mxv = jnp.where(has, mxv, 0.0)
    mnv = jnp.where(has, mnv, 0.0)
    return jnp.concatenate([_tc_elu(mxv), _tc_elu(mean), _tc_elu(mnv)],
                           axis=1)


def _tc_matmul2(w, bias, pre=False):
    kdim = w.shape[0]

    def body(x_ref, w_ref, b_ref, p_ref, q_ref):
        xb = x_ref[...]
        if pre:
            xb = _vrsp_post(xb)
        y = jnp.dot(xb, w_ref[...],
                    preferred_element_type=jnp.float32,
                    precision=lax.Precision.HIGHEST) + b_ref[0:1, :]
        p_ref[...] = y[:, :16]
        q_ref[...] = y[:, 16:32]

    f = pl.pallas_call(
        body,
        grid=(NPAD // _RB,),
        in_specs=[
            pl.BlockSpec((_RB, kdim), lambda i: (i, 0)),
            pl.BlockSpec((kdim, 32), lambda i: (0, 0)),
            pl.BlockSpec((8, 32), lambda i: (0, 0)),
        ],
        out_specs=[
            pl.BlockSpec((_RB, 16), lambda i: (i, 0)),
            pl.BlockSpec((_RB, 16), lambda i: (i, 0)),
        ],
        out_shape=[
            jax.ShapeDtypeStruct((NPAD, 16), jnp.float32),
            jax.ShapeDtypeStruct((NPAD, 16), jnp.float32),
        ],
    )
    return lambda x: f(x, w, bias)


def _tc_gat_prep(wgcat, msrc, mdst):
    def body(h_ref, wg_ref, ms_ref, md_ref, hg_ref, as_ref, ad_ref, es_ref):
        hg = jnp.dot(_vrsp_post(h_ref[...]), wg_ref[...],
                     preferred_element_type=jnp.float32,
                     precision=lax.Precision.HIGHEST)
        a_s = jnp.dot(hg, ms_ref[...], preferred_element_type=jnp.float32,
                      precision=lax.Precision.HIGHEST)
        a_d = jnp.dot(hg, md_ref[...], preferred_element_type=jnp.float32,
                      precision=lax.Precision.HIGHEST)
        as2 = jnp.concatenate([a_s, a_s], axis=1)
        ad2 = jnp.concatenate([a_d, a_d], axis=1)
        ev = as2 + ad2
        hg_ref[...] = hg
        as_ref[...] = as2
        ad_ref[...] = ad2
        es_ref[...] = jnp.where(ev > 0, ev, 0.2 * ev)

    f = pl.pallas_call(
        body,
        grid=(NPAD // _RB,),
        in_specs=[
            pl.BlockSpec((_RB, 48), lambda i: (i, 0)),
            pl.BlockSpec((48, 64), lambda i: (0, 0)),
            pl.BlockSpec((64, 8), lambda i: (0, 0)),
            pl.BlockSpec((64, 8), lambda i: (0, 0)),
        ],
        out_specs=[
            pl.BlockSpec((_RB, 64), lambda i: (i, 0)),
            pl.BlockSpec((_RB, 16), lambda i: (i, 0)),
            pl.BlockSpec((_RB, 16), lambda i: (i, 0)),
            pl.BlockSpec((_RB, 16), lambda i: (i, 0)),
        ],
        out_shape=[
            jax.ShapeDtypeStruct((NPAD, 64), jnp.float32),
            jax.ShapeDtypeStruct((NPAD, 16), jnp.float32),
            jax.ShapeDtypeStruct((NPAD, 16), jnp.float32),
            jax.ShapeDtypeStruct((NPAD, 16), jnp.float32),
        ],
    )
    return lambda h2: f(h2, wgcat, msrc, mdst)


def _tc_final(p8to64, bg64, wo64, bo):
    def body(g_ref, d_ref, p_ref, bg_ref, wo_ref, bo_ref, o_ref):
        dh = jnp.dot(d_ref[...][:, 0:8], p_ref[...],
                     preferred_element_type=jnp.float32,
                     precision=lax.Precision.HIGHEST)
        y = g_ref[...] / dh + bg_ref[0:1, :]
        y = jnp.where(y > 0, y, jnp.exp(jnp.minimum(y, 0.0)) - 1.0)
        o_ref[...] = jnp.dot(y, wo_ref[...],
                             preferred_element_type=jnp.float32,
                             precision=lax.Precision.HIGHEST) + bo_ref[0:1, :]

    f = pl.pallas_call(
        body,
        grid=(NPAD // _RB,),
        in_specs=[
            pl.BlockSpec((_RB, 64), lambda i: (i, 0)),
            pl.BlockSpec((_RB, 16), lambda i: (i, 0)),
            pl.BlockSpec((8, 64), lambda i: (0, 0)),
            pl.BlockSpec((8, 64), lambda i: (0, 0)),
            pl.BlockSpec((64, 8), lambda i: (0, 0)),
            pl.BlockSpec((8, 8), lambda i: (0, 0)),
        ],
        out_specs=pl.BlockSpec((_RB, 8), lambda i: (i, 0)),
        out_shape=jax.ShapeDtypeStruct((NPAD, 8), jnp.float32),
    )
    return lambda g, den: f(g, den, p8to64, bg64, wo64, bo)


def kernel(x, edge_index, W1, b1, W2, b2, Wg, att_src, att_dst, bg, Wo, bo):
    f32 = jnp.float32
    src = edge_index[0]
    dst = edge_index[1]

    x_pad = jnp.zeros((NPAD, D), f32).at[:N].set(x)

    w1cat = jnp.zeros((D, 32), f32)
    w1cat = w1cat.at[:, 0:H0].set(W1[:D])
    w1cat = w1cat.at[:, 16:16 + H0].set(W1[D:])
    bias1 = jnp.zeros((8, 32), f32).at[0, 0:H0].set(b1).at[0, 12].set(1.0)

    w2cat = jnp.zeros((48, 32), f32)
    for blk in range(3):
        rows = W2[blk * H0:(blk + 1) * H0]
        w2cat = w2cat.at[blk * 16:blk * 16 + H0, 0:H1].set(rows[:, :])
    w2s = W2[3 * H0:]
    for blk in range(3):
        rows = w2s[blk * H0:(blk + 1) * H0]
        w2cat = w2cat.at[blk * 16:blk * 16 + H0, 16:16 + H1].set(rows[:, :])
    bias2 = jnp.zeros((8, 32), f32).at[0, 0:H1].set(b2).at[0, 12].set(1.0)

    wgcat = jnp.zeros((48, 64), f32)
    for blk in range(3):
        rows = Wg[blk * H1:(blk + 1) * H1]
        wgcat = wgcat.at[blk * 16:blk * 16 + H1, 0:HEADS * H2].set(rows)
    msrc = jnp.zeros((64, 8), f32)
    mdst = jnp.zeros((64, 8), f32)
    for h in range(HEADS):
        msrc = msrc.at[h * H2:(h + 1) * H2, h].set(att_src[h])
        mdst = mdst.at[h * H2:(h + 1) * H2, h].set(att_dst[h])

    bg64 = jnp.zeros((8, 64), f32).at[0, :HEADS * H2].set(bg)
    wo64 = jnp.zeros((64, 8), f32).at[:HEADS * H2, 0].set(Wo[:, 0])
    bo8 = jnp.zeros((8, 8), f32).at[0, 0].set(bo[0])
    hmap = jnp.minimum(jnp.arange(64) // H2, HEADS - 1)
    p8to64 = (jnp.arange(8)[:, None] == hmap[None, :]).astype(f32)

    raw1, cnts1 = _sc_distribute()(dst, src)
    raw2, cnts2 = _sc_distribute()(src, dst)
    raw1 = raw1.reshape(NT, NT * SUBCAP)
    raw2 = raw2.reshape(NT, NT * SUBCAP)

    p1, q1 = _tc_matmul2(w1cat, bias1)(x_pad)
    h1, lists, counts = _sc_layer1()(raw1, cnts1, p1, q1)

    p2, q2 = _tc_matmul2(w2cat, bias2, pre=True)(h1)
    h2 = _sc_layer2()(lists, counts, p2, q2)

    hg, as2, ad2, es2 = _tc_gat_prep(wgcat, msrc, mdst)(h2)
    gf, denf = _sc_gat()(raw2, cnts2, as2, ad2, es2, hg)

    out = _tc_final(p8to64, bg64, wo64, bo8)(gf.reshape(NPAD, 64),
                                             denf.reshape(NPAD, 16))
    return out[:N, :1]


# speedup vs baseline: 51.8948x; 1.0297x over previous
import functools

import jax
import jax.numpy as jnp
from jax import lax
from jax.experimental import pallas as pl
from jax.experimental.pallas import tpu as pltpu, tpu_sc as plsc

N = 10000
E = 320000
D = 128
H0, H1, H2 = 12, 9, 7
HEADS = 8

NT = 32
NPT = 320
NPAD = NT * NPT
CH = 2560
NCH = E // CH
NBCH = CH // 16
CAP = 16384
LV = CAP + 160
GRP = 64
NEG = -3.0e38

SUBCAP = 640
EPT = E // NT
CHD = 2000
NCHD = EPT // CHD
NBD = CHD // 16
RAWLEN = NT * NT * SUBCAP

_SC_PARAMS = pltpu.CompilerParams(needs_layout_passes=False,
                                  use_tc_tiling_on_sc=False)


def _prefix16(v):
    io = lax.iota(jnp.int32, 16)
    for k in (1, 2, 4, 8):
        sh = jnp.take(v, jnp.maximum(io - k, 0))
        v = v + jnp.where(io >= k, sh, 0)
    return v


def _splat_i(x):
    return jnp.full((16,), x, jnp.int32)


def _row(ref2d, r):
    io = lax.iota(jnp.int32, 16)
    return plsc.load_gather(ref2d, [_splat_i(r), io])


def _scan_edges(key_c, val_c, list_v, base, cnt0):
    dl = key_c - base
    m = (dl >= 0) & (dl < NPT)
    pk = (dl << 14) | val_c
    cs = _prefix16(jnp.where(m, 1, 0))
    cntc = jnp.minimum(cnt0, CAP - 16)
    pos = jnp.where(m, cntc + cs - 1, LV - 1)
    plsc.store_scatter(list_v, [pos], pk)
    return cnt0 + cs[15]


def _pad_list(list_v, cnt):
    io = lax.iota(jnp.int32, 16)
    dummy = jnp.full((16,), NPT << 14, jnp.int32)
    cntc = jnp.minimum(cnt, CAP - 16)
    for k in range(8):
        plsc.store_scatter(list_v, [cntc + k * 16 + io], dummy)
    return (cntc + 127) & ~127


def _prefmax16(v):
    io = lax.iota(jnp.int32, 16)
    for k in (1, 2, 4, 8):
        sh = jnp.take(v, jnp.maximum(io - k, 0))
        v = jnp.maximum(v, jnp.where(io >= k, sh, 0))
    return v


def _sc_distribute(interpret=False):
    mesh = plsc.VectorSubcoreMesh(core_axis_name="c", subcore_axis_name="s")

    @functools.partial(
        pl.kernel,
        out_type=(
            jax.ShapeDtypeStruct((RAWLEN,), jnp.int32),
            jax.ShapeDtypeStruct((NT * 48,), jnp.int32),
        ),
        mesh=mesh,
        interpret=interpret,
        compiler_params=_SC_PARAMS,
        scratch_types=[
            pltpu.VMEM((CHD,), jnp.int32),
            pltpu.VMEM((CHD,), jnp.int32),
            pltpu.VMEM((NT * SUBCAP,), jnp.int32),
            pltpu.VMEM((48,), jnp.int32),
            pltpu.VMEM((48,), jnp.int32),
        ],
    )
    def k(key_hbm, val_hbm, raw_hbm, cnts_hbm, kchunk, vchunk, sub, ctr,
          pcnt):
        wid = _wid()
        estart = wid * EPT
        io = lax.iota(jnp.int32, 16)
        for c in range(3):
            ctr[pl.ds(c * 16, 16)] = jnp.zeros((16,), jnp.int32)

        def chunk_body(ch, _):
            pltpu.sync_copy(key_hbm.at[pl.ds(estart + ch * CHD, CHD)], kchunk)
            pltpu.sync_copy(val_hbm.at[pl.ds(estart + ch * CHD, CHD)], vchunk)

            def batch_body(i, _2):
                d = kchunk[pl.ds(i * 16, 16)]
                s = vchunk[pl.ds(i * 16, 16)]
                b = ((d >> 6) * 205) >> 10
                pk = ((d - b * NPT) << 14) | s
                bs16, pks = plsc.sort_key_val((b << 4) | io, pk)
                bs = bs16 >> 4
                prev = jnp.take(bs, jnp.maximum(io - 1, 0))
                isfirst = (io == 0) | (bs != prev)
                sf = _prefmax16(jnp.where(isfirst, io, 0))
                rank = io - sf
                ctrv = plsc.load_gather(ctr, [bs])
                pos = jnp.minimum(ctrv + rank, SUBCAP - 1)
                nxt = jnp.take(bs, jnp.minimum(io + 1, 15))
                islast = (io == 15) | (bs != nxt)
                plsc.store_scatter(ctr, [jnp.where(islast, bs, 40)], pos + 1)
                plsc.store_scatter(sub, [bs * SUBCAP + pos], pks)
                return 0

            lax.fori_loop(0, NBD, batch_body, 0)
            return 0

        lax.fori_loop(0, NCHD, chunk_body, 0)

        dummy = jnp.full((16,), NPT << 14, jnp.int32)
        c0 = ctr[pl.ds(0, 16)]
        c1 = ctr[pl.ds(16, 16)]
        for bb in range(NT):
            cbs = jnp.minimum((c0 if bb < 16 else c1)[bb % 16], SUBCAP - 16)
            plsc.store_scatter(sub, [bb * SUBCAP + cbs + io], dummy)
            pad = (cbs + 15) & ~15
            plsc.store_scatter(pcnt, [_splat_i(bb)], _splat_i(pad))

        pltpu.sync_copy(sub, raw_hbm.at[pl.ds(wid * NT * SUBCAP,
                                              NT * SUBCAP)])
        pltpu.sync_copy(pcnt, cnts_hbm.at[pl.ds(wid * 48, 48)])

    return k


def _merge_lists(raw_hbm, cnts_hbm, wid, c2d, staging, list_v, sem):
    io = lax.iota(jnp.int32, 16)
    pltpu.sync_copy(cnts_hbm, c2d)
    c16a = plsc.load_gather(c2d, [io * 48 + _splat_i(wid)])
    c16b = plsc.load_gather(c2d, [(io + 16) * 48 + _splat_i(wid)])
    pa = _prefix16(c16a)
    pb = _prefix16(c16b)
    tot_a = pa[15]
    sa = pa - c16a
    sb = pb - c16b + tot_a
    total = tot_a + pb[15]

    for t in range(NT):
        pltpu.sync_copy(raw_hbm.at[t, pl.ds(wid * SUBCAP, SUBCAP)],
                        staging.at[t])

    for t in range(NT):
        cnt_t = (c16a if t < 16 else c16b)[t % 16]
        st = (sa if t < 16 else sb)[t % 16]

        def cp(i, _, t=t, st=st):
            v = plsc.load_gather(staging, [_splat_i(t), i * 16 + io])
            list_v[pl.ds(st + i * 16, 16)] = v
            return 0

        lax.fori_loop(0, cnt_t >> 4, cp, 0)
    return total


def _stream_scan(key_hbm, val_hbm, kchunk, vchunk, list_v, base):
    def chunk_body(ch, cnt):
        pltpu.sync_copy(key_hbm.at[pl.ds(ch * CH, CH)], kchunk)
        pltpu.sync_copy(val_hbm.at[pl.ds(ch * CH, CH)], vchunk)

        def batch_body(i, c):
            kc = kchunk[pl.ds(i * 16, 16)]
            vc = vchunk[pl.ds(i * 16, 16)]
            return _scan_edges(kc, vc, list_v, base, c)

        return lax.fori_loop(0, NBCH, batch_body, cnt)

    cnt = lax.fori_loop(0, NCH, chunk_body, 0)
    return _pad_list(list_v, cnt), cnt


def _unpack_grp(list_v, g, gidx, halfoff):
    def unpack(i, _):
        pkb = list_v[pl.ds(g * GRP + i * 16, 16)]
        gidx[pl.ds(halfoff + i * 16, 16)] = pkb & 16383
        return 0
    lax.fori_loop(0, GRP // 16, unpack, 0)


def _fire_gathers(g, gidx, pairs, sem0, sem1):
    @pl.when((g & 1) == 0)
    def _():
        for (src, dstv) in pairs:
            pltpu.async_copy(src.at[gidx.at[pl.ds(0, GRP)]],
                             dstv.at[pl.ds(0, GRP)], sem0)

    @pl.when((g & 1) == 1)
    def _():
        for (src, dstv) in pairs:
            pltpu.async_copy(src.at[gidx.at[pl.ds(GRP, GRP)]],
                             dstv.at[pl.ds(GRP, GRP)], sem1)


def _drain_gathers(g, pairs, sem0, sem1):
    @pl.when((g & 1) == 0)
    def _():
        for (src, dstv) in pairs:
            pltpu.make_async_copy(src.at[pl.ds(0, GRP)],
                                  dstv.at[pl.ds(0, GRP)], sem0).wait()

    @pl.when((g & 1) == 1)
    def _():
        for (src, dstv) in pairs:
            pltpu.make_async_copy(src.at[pl.ds(0, GRP)],
                                  dstv.at[pl.ds(GRP, GRP)], sem1).wait()


def _db_prologue(list_v, ngroups, gidx, pairs, sem0):
    @pl.when(ngroups > 0)
    def _():
        _unpack_grp(list_v, 0, gidx, 0)
        for (src, dstv) in pairs:
            pltpu.async_copy(src.at[gidx.at[pl.ds(0, GRP)]],
                             dstv.at[pl.ds(0, GRP)], sem0)


def _layer_edges(list_v, ngroups, q_hbm, gidx, qrows, sem0, sem1, pstage,
                 acc_s, acc_mx, acc_mn):
    io = lax.iota(jnp.int32, 16)
    pairs = [(q_hbm, qrows)]
    _db_prologue(list_v, ngroups, gidx, pairs, sem0)

    def group_body(g, _):
        @pl.when(g + 1 < ngroups)
        def _():
            _unpack_grp(list_v, g + 1, gidx, ((g + 1) & 1) * GRP)
            _fire_gathers(g + 1, gidx, pairs, sem0, sem1)
        _drain_gathers(g, pairs, sem0, sem1)
        qoff = (g & 1) * GRP

        def batch_body(i, _2):
            pkb = list_v[pl.ds(g * GRP + i * 16, 16)]
            for j in range(16):
                dsc = pkb[j] >> 14
                rb = dsc * 16
                qv = plsc.load_gather(qrows,
                                      [_splat_i(qoff + i * 16 + j), io])
                pv = pstage[pl.ds(rb, 16)]
                mrow = pv + qv
                plsc.addupdate(acc_s.at[pl.ds(rb, 16)], mrow)
                acc_mx[pl.ds(rb, 16)] = jnp.maximum(acc_mx[pl.ds(rb, 16)], mrow)
                acc_mn[pl.ds(rb, 16)] = jnp.minimum(acc_mn[pl.ds(rb, 16)], mrow)
            return 0

        lax.fori_loop(0, GRP // 16, batch_body, 0)
        return 0

    lax.fori_loop(0, ngroups, group_body, 0)


def _write_raw48(acc_s, acc_mx, acc_mn, h2d):
    io = lax.iota(jnp.int32, 16)

    def node_body(r, _):
        rb = r * 16
        plsc.store_scatter(h2d, [_splat_i(r), io], acc_s[pl.ds(rb, 16)])
        plsc.store_scatter(h2d, [_splat_i(r), 16 + io], acc_mx[pl.ds(rb, 16)])
        plsc.store_scatter(h2d, [_splat_i(r), 32 + io], acc_mn[pl.ds(rb, 16)])
        return 0

    lax.fori_loop(0, NPT, node_body, 0)


def _init_accs(pstage, p_hbm, base, acc_s, acc_mx, acc_mn, p2d):
    pltpu.sync_copy(p_hbm.at[pl.ds(base, NPT)], p2d)
    io = lax.iota(jnp.int32, 16)

    def cp(r, _):
        pstage[pl.ds(r * 16, 16)] = plsc.load_gather(p2d, [_splat_i(r), io])
        return 0
    lax.fori_loop(0, NPT, cp, 0)

    z = jnp.zeros((16,), jnp.float32)
    hi = jnp.full((16,), -NEG, jnp.float32)
    lo = jnp.full((16,), NEG, jnp.float32)

    def zr(r, _):
        acc_s[pl.ds(r * 16, 16)] = z
        acc_mx[pl.ds(r * 16, 16)] = lo
        acc_mn[pl.ds(r * 16, 16)] = hi
        return 0
    lax.fori_loop(0, NPT + 1, zr, 0)


def _write_count(counts_hbm, cbuf, wid, cnt):
    cbuf[pl.ds(0, 16)] = _splat_i(cnt)
    pltpu.sync_copy(cbuf, counts_hbm.at[wid])


def _read_count(counts_hbm, cbuf, wid):
    pltpu.sync_copy(counts_hbm.at[wid], cbuf)
    return cbuf[pl.ds(0, 16)][0]


def _wid():
    return lax.axis_index("s") * 2 + lax.axis_index("c")


def _sc_layer1():
    mesh = plsc.VectorSubcoreMesh(core_axis_name="c", subcore_axis_name="s")

    @functools.partial(
        pl.kernel,
        out_type=(
            jax.ShapeDtypeStruct((NPAD, 48), jnp.float32),
            jax.ShapeDtypeStruct((NT, LV), jnp.int32),
            jax.ShapeDtypeStruct((NT, 16), jnp.int32),
        ),
        mesh=mesh,
        compiler_params=_SC_PARAMS,
        scratch_types=[
            pltpu.VMEM((LV,), jnp.int32),
            pltpu.VMEM((NT * 48,), jnp.int32),
            pltpu.VMEM((NT, SUBCAP), jnp.int32),
            pltpu.VMEM((NPT * 16,), jnp.float32),
            pltpu.VMEM((NPT, 16), jnp.float32),
            pltpu.VMEM(((NPT + 1) * 16,), jnp.float32),
            pltpu.VMEM(((NPT + 1) * 16,), jnp.float32),
            pltpu.VMEM(((NPT + 1) * 16,), jnp.float32),
            pltpu.VMEM((2 * GRP,), jnp.int32),
            pltpu.VMEM((2 * GRP, 16), jnp.float32),
            pltpu.VMEM((NPT, 48), jnp.float32),
            pltpu.VMEM((16,), jnp.int32),
            pltpu.SemaphoreType.DMA,
            pltpu.SemaphoreType.DMA,
        ],
    )
    def k(raw_hbm, cnts_hbm, p_hbm, q_hbm, h_hbm, lists_hbm, counts_hbm,
          list_v, c2d, staging, pstage, p2d, acc_s, acc_mx, acc_mn,
          gidx, qrows, h2d, cbuf, sem0, sem1):
        wid = _wid()
        base = wid * NPT
        cnt = _merge_lists(raw_hbm, cnts_hbm, wid, c2d, staging, list_v, sem0)
        cnt128 = _pad_list(list_v, cnt)
        _write_count(counts_hbm, cbuf, wid, cnt)
        pltpu.sync_copy(list_v, lists_hbm.at[wid])
        _init_accs(pstage, p_hbm, base, acc_s, acc_mx, acc_mn, p2d)
        _layer_edges(list_v, cnt128 >> 6, q_hbm, gidx, qrows, sem0, sem1,
                     pstage, acc_s, acc_mx, acc_mn)
        _write_raw48(acc_s, acc_mx, acc_mn, h2d)
        pltpu.sync_copy(h2d, h_hbm.at[pl.ds(base, NPT)])

    return k


def _sc_layer2():
    mesh = plsc.VectorSubcoreMesh(core_axis_name="c", subcore_axis_name="s")

    @functools.partial(
        pl.kernel,
        out_type=jax.ShapeDtypeStruct((NPAD, 48), jnp.float32),
        mesh=mesh,
        compiler_params=_SC_PARAMS,
        scratch_types=[
            pltpu.VMEM((LV,), jnp.int32),
            pltpu.VMEM((NPT * 16,), jnp.float32),
            pltpu.VMEM((NPT, 16), jnp.float32),
            pltpu.VMEM(((NPT + 1) * 16,), jnp.float32),
            pltpu.VMEM(((NPT + 1) * 16,), jnp.float32),
            pltpu.VMEM(((NPT + 1) * 16,), jnp.float32),
            pltpu.VMEM((2 * GRP,), jnp.int32),
            pltpu.VMEM((2 * GRP, 16), jnp.float32),
            pltpu.VMEM((NPT, 48), jnp.float32),
            pltpu.VMEM((16,), jnp.int32),
            pltpu.SemaphoreType.DMA,
            pltpu.SemaphoreType.DMA,
        ],
    )
    def k(lists_hbm, counts_hbm, p_hbm, q_hbm, h_hbm,
          list_v, pstage, p2d, acc_s, acc_mx, acc_mn, gidx, qrows,
          h2d, cbuf, sem0, sem1):
        wid = _wid()
        base = wid * NPT
        pltpu.sync_copy(lists_hbm.at[wid], list_v)
        cnt = _read_count(counts_hbm, cbuf, wid)
        cnt128 = _pad_list(list_v, cnt)
        _init_accs(pstage, p_hbm, base, acc_s, acc_mx, acc_mn, p2d)
        _layer_edges(list_v, cnt128 >> 6, q_hbm, gidx, qrows, sem0, sem1,
                     pstage, acc_s, acc_mx, acc_mn)
        _write_raw48(acc_s, acc_mx, acc_mn, h2d)
        pltpu.sync_copy(h2d, h_hbm.at[pl.ds(base, NPT)])

    return k


def _head_map(c):
    g = lax.iota(jnp.int32, 16) + c * 16
    return jnp.minimum((g * 2341) >> 14, 7)


def _sc_gat():
    mesh = plsc.VectorSubcoreMesh(core_axis_name="c", subcore_axis_name="s")

    @functools.partial(
        pl.kernel,
        out_type=(
            jax.ShapeDtypeStruct((NPAD * 64,), jnp.float32),
            jax.ShapeDtypeStruct((NPAD * 16,), jnp.float32),
        ),
        mesh=mesh,
        compiler_params=_SC_PARAMS,
        scratch_types=[
            pltpu.VMEM((LV,), jnp.int32),
            pltpu.VMEM((NT * 48,), jnp.int32),
            pltpu.VMEM((NT, SUBCAP), jnp.int32),
            pltpu.VMEM((NPT, 16), jnp.float32),
            pltpu.VMEM((NPT, 16), jnp.float32),
            pltpu.VMEM((NPT, 64), jnp.float32),
            pltpu.VMEM(((NPT + 1) * 16,), jnp.float32),
            pltpu.VMEM(((NPT + 1) * 16,), jnp.float32),
            pltpu.VMEM(((NPT + 1) * 64,), jnp.float32),
            pltpu.VMEM((2 * GRP,), jnp.int32),
            pltpu.VMEM((2 * GRP, 16), jnp.float32),
            pltpu.VMEM((2 * GRP, 64), jnp.float32),
            pltpu.SemaphoreType.DMA,
            pltpu.SemaphoreType.DMA,
        ],
    )
    def k(raw_hbm, cnts_hbm, as2_hbm, ad2_hbm, es2_hbm, hg_hbm, g_hbm,
          den_hbm, list_v, c2d, staging, ad2d, es2d, hgown, emax, den,
          gacc, gidx, asrows, hgrows, sem0, sem1):
        wid = _wid()
        base = wid * NPT
        io = lax.iota(jnp.int32, 16)

        cnt = _merge_lists(raw_hbm, cnts_hbm, wid, c2d, staging, list_v, sem0)
        cnt128 = _pad_list(list_v, cnt)
        ngroups = cnt128 >> 6

        pltpu.sync_copy(ad2_hbm.at[pl.ds(base, NPT)], ad2d)
        pltpu.sync_copy(es2_hbm.at[pl.ds(base, NPT)], es2d)
        pltpu.sync_copy(hg_hbm.at[pl.ds(base, NPT)], hgown)

        def init_emax(r, _):
            emax[pl.ds(r * 16, 16)] = _row(es2d, r)
            return 0
        lax.fori_loop(0, NPT, init_emax, 0)
        emax[pl.ds(NPT * 16, 16)] = jnp.full((16,), NEG, jnp.float32)

        pairs1 = [(as2_hbm, asrows)]
        _db_prologue(list_v, ngroups, gidx, pairs1, sem0)

        def p1_group(g, _):
            @pl.when(g + 1 < ngroups)
            def _():
                _unpack_grp(list_v, g + 1, gidx, ((g + 1) & 1) * GRP)
                _fire_gathers(g + 1, gidx, pairs1, sem0, sem1)
            _drain_gathers(g, pairs1, sem0, sem1)
            aoff = (g & 1) * GRP

            def p1_batch(i, _2):
                pkb = list_v[pl.ds(g * GRP + i * 16, 16)]
                for j in range(16):
                    il = pkb[j] >> 14
                    rb = il * 16
                    av = plsc.load_gather(asrows,
                                          [_splat_i(aoff + i * 16 + j), io])
                    ev = av + _row(ad2d, il)
                    el = jnp.where(ev > 0, ev, 0.2 * ev)
                    emax[pl.ds(rb, 16)] = jnp.maximum(emax[pl.ds(rb, 16)], el)
                return 0

            lax.fori_loop(0, GRP // 16, p1_batch, 0)
            return 0

        lax.fori_loop(0, ngroups, p1_group, 0)

        hmaps = [_head_map(c) for c in range(4)]

        def init_self(r, _):
            al = jnp.exp(_row(es2d, r) - emax[pl.ds(r * 16, 16)])
            den[pl.ds(r * 16, 16)] = al
            for c in range(4):
                ah = jnp.take(al, hmaps[c])
                hv = plsc.load_gather(hgown, [_splat_i(r), c * 16 + io])
                gacc[pl.ds(r * 64 + c * 16, 16)] = ah * hv
            return 0
        lax.fori_loop(0, NPT, init_self, 0)
        den[pl.ds(NPT * 16, 16)] = jnp.full((16,), 1.0, jnp.float32)

        pairs2 = [(as2_hbm, asrows), (hg_hbm, hgrows)]
        _db_prologue(list_v, ngroups, gidx, pairs2, sem0)

        def p2_group(g, _):
            @pl.when(g + 1 < ngroups)
            def _():
                _unpack_grp(list_v, g + 1, gidx, ((g + 1) & 1) * GRP)
                _fire_gathers(g + 1, gidx, pairs2, sem0, sem1)
            _drain_gathers(g, pairs2, sem0, sem1)
            aoff = (g & 1) * GRP

            def p2_batch(i, _2):
                pkb = list_v[pl.ds(g * GRP + i * 16, 16)]
                for j in range(16):
                    il = pkb[j] >> 14
                    rb = il * 16
                    av = plsc.load_gather(asrows,
                                          [_splat_i(aoff + i * 16 + j), io])
                    ev = av + _row(ad2d, il)
                    el = jnp.where(ev > 0, ev, 0.2 * ev)
                    al = jnp.exp(el - emax[pl.ds(rb, 16)])
                    plsc.addupdate(den.at[pl.ds(rb, 16)], al)
                    gb = il * 64
                    for c in range(4):
                        ah = jnp.take(al, hmaps[c])
                        hv = plsc.load_gather(
                            hgrows, [_splat_i(aoff + i * 16 + j), c * 16 + io])
                        o = pl.ds(gb + c * 16, 16)
                        plsc.addupdate(gacc.at[o], ah * hv)
                return 0

            lax.fori_loop(0, GRP // 16, p2_batch, 0)
            return 0

        lax.fori_loop(0, ngroups, p2_group, 0)

        pltpu.sync_copy(gacc.at[pl.ds(0, NPT * 64)],
                        g_hbm.at[pl.ds(base * 64, NPT * 64)])
        pltpu.sync_copy(den.at[pl.ds(0, NPT * 16)],
                        den_hbm.at[pl.ds(base * 16, NPT * 16)])

    return k


_RB = 1024


def _tc_elu(v):
    return jnp.where(v > 0, v, jnp.exp(jnp.minimum(v, 0.0)) - 1.0)


def _vrsp_post(hb):
    s = hb[:, 0:16]
    mxv = hb[:, 16:32]
    mnv = hb[:, 32:48]
    cnt = s[:, 12:13]
    has = cnt > 0
    mean = jnp.where(has, s / jnp.maximum(cnt, 1.0), 0.0)
    mxv = jnp.where(has, mxv, 0.0)
    mnv = jnp.where(has, mnv, 0.0)
    return jnp.concatenate([_tc_elu(mxv), _tc_elu(mean), _tc_elu(mnv)],
                           axis=1)


def _tc_matmul2(w, bias, pre=False):
    kdim = w.shape[0]

    def body(x_ref, w_ref, b_ref, p_ref, q_ref):
        xb = x_ref[...]
        if pre:
            xb = _vrsp_post(xb)
        y = jnp.dot(xb, w_ref[...],
                    preferred_element_type=jnp.float32,
                    precision=lax.Precision.HIGHEST) + b_ref[0:1, :]
        p_ref[...] = y[:, :16]
        q_ref[...] = y[:, 16:32]

    f = pl.pallas_call(
        body,
        grid=(NPAD // _RB,),
        in_specs=[
            pl.BlockSpec((_RB, kdim), lambda i: (i, 0)),
            pl.BlockSpec((kdim, 32), lambda i: (0, 0)),
            pl.BlockSpec((8, 32), lambda i: (0, 0)),
        ],
        out_specs=[
            pl.BlockSpec((_RB, 16), lambda i: (i, 0)),
            pl.BlockSpec((_RB, 16), lambda i: (i, 0)),
        ],
        out_shape=[
            jax.ShapeDtypeStruct((NPAD, 16), jnp.float32),
            jax.ShapeDtypeStruct((NPAD, 16), jnp.float32),
        ],
    )
    return lambda x: f(x, w, bias)


def _tc_gat_prep(wgcat, msrc, mdst):
    def body(h_ref, wg_ref, ms_ref, md_ref, hg_ref, as_ref, ad_ref, es_ref):
        hg = jnp.dot(_vrsp_post(h_ref[...]), wg_ref[...],
                     preferred_element_type=jnp.float32,
                     precision=lax.Precision.HIGHEST)
        a_s = jnp.dot(hg, ms_ref[...], preferred_element_type=jnp.float32,
                      precision=lax.Precision.HIGHEST)
        a_d = jnp.dot(hg, md_ref[...], preferred_element_type=jnp.float32,
                      precision=lax.Precision.HIGHEST)
        as2 = jnp.concatenate([a_s, a_s], axis=1)
        ad2 = jnp.concatenate([a_d, a_d], axis=1)
        ev = as2 + ad2
        hg_ref[...] = hg
        as_ref[...] = as2
        ad_ref[...] = ad2
        es_ref[...] = jnp.where(ev > 0, ev, 0.2 * ev)

    f = pl.pallas_call(
        body,
        grid=(NPAD // _RB,),
        in_specs=[
            pl.BlockSpec((_RB, 48), lambda i: (i, 0)),
            pl.BlockSpec((48, 64), lambda i: (0, 0)),
            pl.BlockSpec((64, 8), lambda i: (0, 0)),
            pl.BlockSpec((64, 8), lambda i: (0, 0)),
        ],
        out_specs=[
            pl.BlockSpec((_RB, 64), lambda i: (i, 0)),
            pl.BlockSpec((_RB, 16), lambda i: (i, 0)),
            pl.BlockSpec((_RB, 16), lambda i: (i, 0)),
            pl.BlockSpec((_RB, 16), lambda i: (i, 0)),
        ],
        out_shape=[
            jax.ShapeDtypeStruct((NPAD, 64), jnp.float32),
            jax.ShapeDtypeStruct((NPAD, 16), jnp.float32),
            jax.ShapeDtypeStruct((NPAD, 16), jnp.float32),
            jax.ShapeDtypeStruct((NPAD, 16), jnp.float32),
        ],
    )
    return lambda h2: f(h2, wgcat, msrc, mdst)


def _tc_final(p8to64, bg64, wo64, bo):
    def body(g_ref, d_ref, p_ref, bg_ref, wo_ref, bo_ref, o_ref):
        dh = jnp.dot(d_ref[...][:, 0:8], p_ref[...],
                     preferred_element_type=jnp.float32,
                     precision=lax.Precision.HIGHEST)
        y = g_ref[...] / dh + bg_ref[0:1, :]
        y = jnp.where(y > 0, y, jnp.exp(jnp.minimum(y, 0.0)) - 1.0)
        o_ref[...] = jnp.dot(y, wo_ref[...],
                             preferred_element_type=jnp.float32,
                             precision=lax.Precision.HIGHEST) + bo_ref[0:1, :]

    f = pl.pallas_call(
        body,
        grid=(NPAD // _RB,),
        in_specs=[
            pl.BlockSpec((_RB, 64), lambda i: (i, 0)),
            pl.BlockSpec((_RB, 16), lambda i: (i, 0)),
            pl.BlockSpec((8, 64), lambda i: (0, 0)),
            pl.BlockSpec((8, 64), lambda i: (0, 0)),
            pl.BlockSpec((64, 8), lambda i: (0, 0)),
            pl.BlockSpec((8, 8), lambda i: (0, 0)),
        ],
        out_specs=pl.BlockSpec((_RB, 8), lambda i: (i, 0)),
        out_shape=jax.ShapeDtypeStruct((NPAD, 8), jnp.float32),
    )
    return lambda g, den: f(g, den, p8to64, bg64, wo64, bo)


def kernel(x, edge_index, W1, b1, W2, b2, Wg, att_src, att_dst, bg, Wo, bo):
    f32 = jnp.float32
    src = edge_index[0]
    dst = edge_index[1]

    x_pad = jnp.zeros((NPAD, D), f32).at[:N].set(x)

    w1cat = jnp.zeros((D, 32), f32)
    w1cat = w1cat.at[:, 0:H0].set(W1[:D])
    w1cat = w1cat.at[:, 16:16 + H0].set(W1[D:])
    bias1 = jnp.zeros((8, 32), f32).at[0, 0:H0].set(b1).at[0, 12].set(1.0)

    w2cat = jnp.zeros((48, 32), f32)
    for blk in range(3):
        rows = W2[blk * H0:(blk + 1) * H0]
        w2cat = w2cat.at[blk * 16:blk * 16 + H0, 0:H1].set(rows[:, :])
    w2s = W2[3 * H0:]
    for blk in range(3):
        rows = w2s[blk * H0:(blk + 1) * H0]
        w2cat = w2cat.at[blk * 16:blk * 16 + H0, 16:16 + H1].set(rows[:, :])
    bias2 = jnp.zeros((8, 32), f32).at[0, 0:H1].set(b2).at[0, 12].set(1.0)

    wgcat = jnp.zeros((48, 64), f32)
    for blk in range(3):
        rows = Wg[blk * H1:(blk + 1) * H1]
        wgcat = wgcat.at[blk * 16:blk * 16 + H1, 0:HEADS * H2].set(rows)
    msrc = jnp.zeros((64, 8), f32)
    mdst = jnp.zeros((64, 8), f32)
    for h in range(HEADS):
        msrc = msrc.at[h * H2:(h + 1) * H2, h].set(att_src[h])
        mdst = mdst.at[h * H2:(h + 1) * H2, h].set(att_dst[h])

    bg64 = jnp.zeros((8, 64), f32).at[0, :HEADS * H2].set(bg)
    wo64 = jnp.zeros((64, 8), f32).at[:HEADS * H2, 0].set(Wo[:, 0])
    bo8 = jnp.zeros((8, 8), f32).at[0, 0].set(bo[0])
    hmap = jnp.minimum(jnp.arange(64) // H2, HEADS - 1)
    p8to64 = (jnp.arange(8)[:, None] == hmap[None, :]).astype(f32)

    raw1, cnts1 = _sc_distribute()(dst, src)
    raw2, cnts2 = _sc_distribute()(src, dst)
    raw1 = raw1.reshape(NT, NT * SUBCAP)
    raw2 = raw2.reshape(NT, NT * SUBCAP)

    p1, q1 = _tc_matmul2(w1cat, bias1)(x_pad)
    h1, lists, counts = _sc_layer1()(raw1, cnts1, p1, q1)

    p2, q2 = _tc_matmul2(w2cat, bias2, pre=True)(h1)
    h2 = _sc_layer2()(lists, counts, p2, q2)

    hg, as2, ad2, es2 = _tc_gat_prep(wgcat, msrc, mdst)(h2)
    gf, denf = _sc_gat()(raw2, cnts2, as2, ad2, es2, hg)

    out = _tc_final(p8to64, bg64, wo64, bo8)(gf.reshape(NPAD, 64),
                                             denf.reshape(NPAD, 16))
    return out[:N, :1]
